# baseline (speedup 1.0000x reference)
import jax
import jax.numpy as jnp
from jax.experimental import pallas as pl
from jax.experimental.pallas import tpu as pltpu

_B = 16
_Q = 20000
_ROWS = _B * _Q
_BLK = 1000
_SCALE = 1.0 / 512.0


def _body(in_ref, out_ref):
    i = pl.program_id(0)
    batch_f = ((i * _BLK) // _Q).astype(jnp.float32)
    v = in_ref[:]
    s = v * _SCALE
    x1 = s[:, 1:2]
    y1 = s[:, 2:3]
    x2 = s[:, 3:4]
    y2 = s[:, 4:5]
    xmn = jnp.clip(jnp.minimum(x1, x2), 0.0, 1.0)
    ymn = jnp.clip(jnp.minimum(y1, y2), 0.0, 1.0)
    xmx = jnp.clip(jnp.maximum(x1, x2), 0.0, 1.0)
    ymx = jnp.clip(jnp.maximum(y1, y2), 0.0, 1.0)
    bcol = jnp.full((_BLK, 1), batch_f, dtype=jnp.float32)
    out_ref[:] = jnp.concatenate([bcol, xmn, ymn, xmx, ymx], axis=1)


def kernel(vit_output, input_images_or_features):
    del input_images_or_features
    flat = vit_output.reshape(_ROWS, 8)
    return pl.pallas_call(
        _body,
        grid=(_ROWS // _BLK,),
        in_specs=[pl.BlockSpec((_BLK, 8), lambda i: (i, 0))],
        out_specs=pl.BlockSpec((_BLK, 5), lambda i: (i, 0)),
        out_shape=jax.ShapeDtypeStruct((_ROWS, 5), jnp.float32),
    )(flat)
